# baseline (speedup 1.0000x reference)
import functools

import jax
import jax.numpy as jnp
from jax import lax
from jax.experimental import pallas as pl
from jax.experimental.pallas import tpu as pltpu
from jax.experimental.pallas import tpu_sc as plsc

BATCH = 32
SEQ_LEN = 8192
D = 128
TOTAL = BATCH * SEQ_LEN
NUM_CORES = 2
NUM_SUBCORES = 16
NW = NUM_CORES * NUM_SUBCORES
BPW = TOTAL // NW
CH = 128
NCH = BPW // CH

_mesh = plsc.VectorSubcoreMesh(core_axis_name="c", subcore_axis_name="s")


@functools.partial(
    pl.kernel,
    mesh=_mesh,
    out_type=jax.ShapeDtypeStruct((TOTAL, D), jnp.float32),
    scratch_types=[
        pltpu.VMEM((NCH, CH), jnp.int32),
        pltpu.VMEM((CH, D), jnp.float32),
        pltpu.SemaphoreType.DMA,
    ],
)
def _emb_lookup(x_hbm, table_hbm, out_hbm, idx_v, rows_v, sem):
    wid = lax.axis_index("s") * NUM_CORES + lax.axis_index("c")
    base = wid * BPW

    pltpu.sync_copy(x_hbm.at[pl.ds(wid * NCH, NCH)], idx_v)

    def body(j, _):
        pltpu.async_copy(table_hbm.at[idx_v.at[j]], rows_v, sem).wait()
        pltpu.sync_copy(rows_v, out_hbm.at[pl.ds(base + j * CH, CH)])
        return ()

    lax.fori_loop(0, NCH, body, ())


def kernel(x, table):
    x2 = x.reshape(TOTAL // CH, CH).astype(jnp.int32)
    out = _emb_lookup(x2, table)
    return out.reshape(BATCH, SEQ_LEN, D)


# speedup vs baseline: 25.3795x; 25.3795x over previous
import functools

import jax
import jax.numpy as jnp
from jax import lax
from jax.experimental import pallas as pl
from jax.experimental.pallas import tpu as pltpu
from jax.experimental.pallas import tpu_sc as plsc

BATCH = 32
SEQ_LEN = 8192
D = 128
NUM_EMB = 6
TOTAL = BATCH * SEQ_LEN
NUM_CORES = 2
NUM_SUBCORES = 16
NW = NUM_CORES * NUM_SUBCORES
BPW = TOTAL // NW
CH = 128
NCH = BPW // CH
NBUF = 2

_mesh = plsc.VectorSubcoreMesh(core_axis_name="c", subcore_axis_name="s")


@functools.partial(
    pl.kernel,
    mesh=_mesh,
    out_type=jax.ShapeDtypeStruct((TOTAL, D), jnp.float32),
    scratch_types=[
        pltpu.VMEM((NCH, CH), jnp.int32),
        pltpu.VMEM_SHARED((NUM_EMB, D), jnp.float32),
        pltpu.VMEM((NBUF, CH, D), jnp.float32),
        pltpu.SemaphoreType.DMA,
        pltpu.SemaphoreType.DMA,
        pltpu.SemaphoreType.DMA,
        pltpu.SemaphoreType.DMA,
    ],
)
def _emb_lookup(x_hbm, table_hbm, out_hbm, idx_v, tab_v, rows_v,
                gsem0, gsem1, wsem0, wsem1):
    wid = lax.axis_index("s") * NUM_CORES + lax.axis_index("c")
    base = wid * BPW
    gsem = (gsem0, gsem1)
    wsem = (wsem0, wsem1)

    @pl.when(lax.axis_index("s") == 0)
    def _():
        pltpu.sync_copy(table_hbm, tab_v)

    pltpu.sync_copy(x_hbm.at[pl.ds(wid * NCH, NCH)], idx_v)
    plsc.subcore_barrier()

    for b in range(NBUF):
        pltpu.async_copy(tab_v.at[idx_v.at[b]], rows_v.at[b], gsem[b])

    def body(j, _):
        for b in range(NBUF):
            jj = j + b
            pltpu.make_async_copy(tab_v.at[idx_v.at[jj]], rows_v.at[b],
                                  gsem[b]).wait()
            pltpu.async_copy(rows_v.at[b],
                             out_hbm.at[pl.ds(base + jj * CH, CH)], wsem[b])
            pltpu.make_async_copy(
                rows_v.at[b], out_hbm.at[pl.ds(base + jj * CH, CH)],
                wsem[b]).wait()

            @pl.when(jj + NBUF < NCH)
            def _():
                pltpu.async_copy(tab_v.at[idx_v.at[jj + NBUF]],
                                 rows_v.at[b], gsem[b])
        return ()

    lax.fori_loop(0, NCH // NBUF, lambda i, c: body(i * NBUF, c), (),
                  unroll=False)


def kernel(x, table):
    x2 = x.reshape(TOTAL // CH, CH).astype(jnp.int32)
    out = _emb_lookup(x2, table)
    return out.reshape(BATCH, SEQ_LEN, D)
